# baseline (speedup 1.0000x reference)
import jax
import jax.numpy as jnp
import numpy as np
from jax.experimental import pallas as pl
from jax.experimental.pallas import tpu as pltpu

_LOG_2PI = float(np.log(2.0 * np.pi))


def _prep_body(L_ref, mu_ref, w_ref, At_ref, Bb_ref):
    K, D, _ = L_ref.shape
    C = K * D
    r2 = jax.lax.broadcasted_iota(jnp.int32, (D, D), 0)
    c2 = jax.lax.broadcasted_iota(jnp.int32, (D, D), 1)
    tril_m = r2 >= c2
    eye2 = (r2 == c2).astype(jnp.float32)
    sig_list = []
    for k in range(K):
        Lt = jnp.where(tril_m, L_ref[k], 0.0)
        Sig = jax.lax.dot_general(Lt, Lt, (((1,), (1,)), ((), ())),
                                  preferred_element_type=jnp.float32) + eye2
        sig_list.append(Sig)
    M3 = jnp.stack(sig_list, axis=1)

    rowi = jax.lax.broadcasted_iota(jnp.int32, (D, 1, 1), 0)
    coli = jax.lax.broadcasted_iota(jnp.int32, (1, 1, D), 2)

    def body(i, carry):
        M, ld = carry
        rm = rowi == i
        cm = coli == i
        r = jnp.sum(jnp.where(rm, M, 0.0), axis=0, keepdims=True)
        c = jnp.sum(jnp.where(cm, M, 0.0), axis=2, keepdims=True)
        p = jnp.sum(jnp.where(cm, r, 0.0), axis=2, keepdims=True)
        pinv = 1.0 / p
        rp = r * pinv
        Mu = M - c * rp
        rowpatch = jnp.where(cm, pinv, rp)
        Mn = jnp.where(rm, rowpatch, jnp.where(cm, -c * pinv, Mu))
        return Mn, ld + jnp.log(p)

    M3, ld = jax.lax.fori_loop(
        0, D, body, (M3, jnp.zeros((1, K, 1), jnp.float32)))

    A2 = M3.reshape(D, C)
    ld2 = ld.reshape(1, K)
    Bm = jnp.sum(M3 * mu_ref[...].reshape(1, K, D), axis=2)
    BtMu = jax.lax.dot_general(Bm, mu_ref[...], (((0,), (1,)), ((), ())),
                               preferred_element_type=jnp.float32)
    kk1 = jax.lax.broadcasted_iota(jnp.int32, (K, K), 0)
    kk2 = jax.lax.broadcasted_iota(jnp.int32, (K, K), 1)
    cdiag = jnp.sum(jnp.where(kk1 == kk2, BtMu, 0.0), axis=0, keepdims=True)
    w = w_ref[...]
    wm = jnp.max(w)
    logw = w - (wm + jnp.log(jnp.sum(jnp.exp(w - wm))))
    beta = -0.5 * (D * _LOG_2PI + ld2) + logw - 0.5 * cdiag

    At_ref[...] = jnp.concatenate(
        [-0.5 * A2, jnp.zeros((D, C), jnp.float32)], axis=0)
    Bb_ref[...] = jnp.concatenate(
        [Bm, beta, jnp.zeros((D - 1, K), jnp.float32)], axis=0)


def _maha_body(X_ref, At_ref, Bb_ref, m_ref, s_ref):
    j = pl.program_id(1)
    nj = pl.num_programs(1)
    Xb = X_ref[...]
    BN, D = Xb.shape
    KD = At_ref.shape[1]
    K = KD // D
    ones_lane = (jax.lax.broadcasted_iota(jnp.int32, (BN, D), 1) == 0)
    Xaug = jnp.concatenate(
        [Xb, jnp.where(ones_lane, 1.0, 0.0)], axis=1)
    ey = (jax.lax.broadcasted_iota(jnp.int32, (D, D), 0) ==
          jax.lax.broadcasted_iota(jnp.int32, (D, D), 1)).astype(jnp.float32)
    Xt = jax.lax.dot_general(ey, Xb, (((1,), (1,)), ((), ())),
                             preferred_element_type=jnp.float32)
    Tt = jax.lax.dot_general(At_ref[...], Xaug, (((0,), (1,)), ((), ())),
                             preferred_element_type=jnp.float32)
    lb = jax.lax.dot_general(Bb_ref[...], Xaug, (((0,), (1,)), ((), ())),
                             preferred_element_type=jnp.float32)
    Z = (Tt.reshape(K, D, BN) * Xt.reshape(1, D, BN)).reshape(KD, BN)
    sel = (jax.lax.broadcasted_iota(jnp.int32, (K, KD), 0) ==
           jax.lax.broadcasted_iota(jnp.int32, (K, KD), 1) // D).astype(jnp.float32)
    qT = jax.lax.dot_general(sel, Z, (((1,), (0,)), ((), ())),
                             preferred_element_type=jnp.float32)
    logp = lb + qT
    mk = jnp.max(logp, axis=0, keepdims=True)
    ss = jnp.sum(jnp.exp(logp - mk), axis=0, keepdims=True)

    @pl.when(j == 0)
    def _():
        m_ref[...] = mk.reshape(1, 1, BN)
        s_ref[...] = ss.reshape(1, 1, BN)

    @pl.when(j > 0)
    def _():
        mp = m_ref[...].reshape(1, BN)
        sp = s_ref[...].reshape(1, BN)
        mn = jnp.maximum(mp, mk)
        s_ref[...] = (sp * jnp.exp(mp - mn) + ss * jnp.exp(mk - mn)).reshape(1, 1, BN)
        m_ref[...] = mn.reshape(1, 1, BN)

    @pl.when(j == nj - 1)
    def _():
        mv = m_ref[...].reshape(1, BN)
        sv = s_ref[...].reshape(1, BN)
        mtot = jnp.max(mv)
        stot = jnp.sum(sv * jnp.exp(mv - mtot))
        m_ref[...] = jnp.full((1, 1, BN), mtot, jnp.float32)
        s_ref[...] = jnp.full((1, 1, BN), stot, jnp.float32)


def kernel(X, mu, L, weights, it):
    N, D = X.shape
    K = mu.shape[0]
    w2 = weights.reshape(1, K)
    At, Bb = pl.pallas_call(
        _prep_body,
        out_shape=[jax.ShapeDtypeStruct((2 * D, K * D), jnp.float32),
                   jax.ShapeDtypeStruct((2 * D, K), jnp.float32)],
    )(L, mu, w2)

    BN = 512
    PAR = 8
    SEQ = N // (PAR * BN)
    m, s = pl.pallas_call(
        _maha_body,
        grid=(PAR, SEQ),
        in_specs=[pl.BlockSpec((BN, D), lambda i, j: (i * SEQ + j, 0)),
                  pl.BlockSpec((2 * D, K * D), lambda i, j: (0, 0)),
                  pl.BlockSpec((2 * D, K), lambda i, j: (0, 0))],
        out_specs=[pl.BlockSpec((1, 1, BN), lambda i, j: (i, 0, 0)),
                   pl.BlockSpec((1, 1, BN), lambda i, j: (i, 0, 0))],
        out_shape=[jax.ShapeDtypeStruct((PAR, 1, BN), jnp.float32),
                   jax.ShapeDtypeStruct((PAR, 1, BN), jnp.float32)],
        compiler_params=pltpu.CompilerParams(
            dimension_semantics=("parallel", "arbitrary")),
    )(X, At, Bb)

    mv = m[:, 0, 0]
    sv = s[:, 0, 0]
    Mx = jnp.max(mv)
    return -(Mx + jnp.log(jnp.sum(sv * jnp.exp(mv - Mx))))


# speedup vs baseline: 1.2527x; 1.2527x over previous
import jax
import jax.numpy as jnp
import numpy as np
from jax.experimental import pallas as pl
from jax.experimental.pallas import tpu as pltpu

_LOG_2PI = float(np.log(2.0 * np.pi))


def _prep_body(L_ref, mu_ref, w_ref, At_ref, Bb_ref):
    K, D, _ = L_ref.shape
    C = K * D
    r2 = jax.lax.broadcasted_iota(jnp.int32, (D, D), 0)
    c2 = jax.lax.broadcasted_iota(jnp.int32, (D, D), 1)
    tril_m = r2 >= c2
    eye2 = (r2 == c2).astype(jnp.float32)
    sig_list = []
    for k in range(K):
        Lt = jnp.where(tril_m, L_ref[k], 0.0)
        Sig = jax.lax.dot_general(Lt, Lt, (((1,), (1,)), ((), ())),
                                  preferred_element_type=jnp.float32) + eye2
        sig_list.append(Sig)
    M3 = jnp.stack(sig_list, axis=1)

    rowi = jax.lax.broadcasted_iota(jnp.int32, (D, 1, 1), 0)
    coli = jax.lax.broadcasted_iota(jnp.int32, (1, 1, D), 2)

    def body(i, carry):
        M, ld = carry
        rm = rowi == i
        cm = coli == i
        r = jnp.sum(jnp.where(rm, M, 0.0), axis=0, keepdims=True)
        c = jnp.sum(jnp.where(cm, M, 0.0), axis=2, keepdims=True)
        p = jnp.sum(jnp.where(cm, r, 0.0), axis=2, keepdims=True)
        pinv = 1.0 / p
        rp = r * pinv
        Mu = M - c * rp
        rowpatch = jnp.where(cm, pinv, rp)
        Mn = jnp.where(rm, rowpatch, jnp.where(cm, -c * pinv, Mu))
        return Mn, ld + jnp.log(p)

    M3, ld = jax.lax.fori_loop(
        0, D, body, (M3, jnp.zeros((1, K, 1), jnp.float32)))

    A2 = M3.reshape(D, C)
    ld2 = ld.reshape(1, K)
    Bm = jnp.sum(M3 * mu_ref[...].reshape(1, K, D), axis=2)
    BtMu = jax.lax.dot_general(Bm, mu_ref[...], (((0,), (1,)), ((), ())),
                               preferred_element_type=jnp.float32)
    kk1 = jax.lax.broadcasted_iota(jnp.int32, (K, K), 0)
    kk2 = jax.lax.broadcasted_iota(jnp.int32, (K, K), 1)
    cdiag = jnp.sum(jnp.where(kk1 == kk2, BtMu, 0.0), axis=0, keepdims=True)
    w = w_ref[...]
    wm = jnp.max(w)
    logw = w - (wm + jnp.log(jnp.sum(jnp.exp(w - wm))))
    beta = -0.5 * (D * _LOG_2PI + ld2) + logw - 0.5 * cdiag

    At_ref[...] = jnp.concatenate(
        [-0.5 * A2, jnp.zeros((D, C), jnp.float32)], axis=0)
    Bb_ref[...] = jnp.concatenate(
        [Bm, beta, jnp.zeros((D - 1, K), jnp.float32)], axis=0)


def _maha_body(X_ref, At_ref, Bb_ref, m_ref, s_ref):
    j = pl.program_id(1)
    nj = pl.num_programs(1)
    Xb = X_ref[...]
    BN, D = Xb.shape
    KD = At_ref.shape[1]
    K = KD // D
    ones_lane = (jax.lax.broadcasted_iota(jnp.int32, (BN, D), 1) == 0)
    Xaug = jnp.concatenate(
        [Xb, jnp.where(ones_lane, 1.0, 0.0)], axis=1)
    ey = (jax.lax.broadcasted_iota(jnp.int32, (D, D), 0) ==
          jax.lax.broadcasted_iota(jnp.int32, (D, D), 1)).astype(jnp.float32)
    Xt = jax.lax.dot_general(ey, Xb, (((1,), (1,)), ((), ())),
                             preferred_element_type=jnp.float32)
    Tt = jax.lax.dot_general(At_ref[...], Xaug, (((0,), (1,)), ((), ())),
                             preferred_element_type=jnp.float32)
    lb = jax.lax.dot_general(Bb_ref[...], Xaug, (((0,), (1,)), ((), ())),
                             preferred_element_type=jnp.float32)
    Tq = Tt.reshape(K, D, BN)
    qT = jnp.sum(Tq * Xt.reshape(1, D, BN), axis=1)
    logp = lb + qT
    mk = jnp.max(logp, axis=0, keepdims=True)
    ss = jnp.sum(jnp.exp(logp - mk), axis=0, keepdims=True)

    @pl.when(j == 0)
    def _():
        m_ref[...] = mk.reshape(1, 1, BN)
        s_ref[...] = ss.reshape(1, 1, BN)

    @pl.when(j > 0)
    def _():
        mp = m_ref[...].reshape(1, BN)
        sp = s_ref[...].reshape(1, BN)
        mn = jnp.maximum(mp, mk)
        s_ref[...] = (sp * jnp.exp(mp - mn) + ss * jnp.exp(mk - mn)).reshape(1, 1, BN)
        m_ref[...] = mn.reshape(1, 1, BN)

    @pl.when(j == nj - 1)
    def _():
        mv = m_ref[...].reshape(1, BN)
        sv = s_ref[...].reshape(1, BN)
        mtot = jnp.max(mv)
        stot = jnp.sum(sv * jnp.exp(mv - mtot))
        m_ref[...] = jnp.full((1, 1, BN), mtot, jnp.float32)
        s_ref[...] = jnp.full((1, 1, BN), stot, jnp.float32)


def kernel(X, mu, L, weights, it):
    N, D = X.shape
    K = mu.shape[0]
    w2 = weights.reshape(1, K)
    At, Bb = pl.pallas_call(
        _prep_body,
        out_shape=[jax.ShapeDtypeStruct((2 * D, K * D), jnp.float32),
                   jax.ShapeDtypeStruct((2 * D, K), jnp.float32)],
    )(L, mu, w2)

    BN = 1024
    PAR = 8
    SEQ = N // (PAR * BN)
    m, s = pl.pallas_call(
        _maha_body,
        grid=(PAR, SEQ),
        in_specs=[pl.BlockSpec((BN, D), lambda i, j: (i * SEQ + j, 0)),
                  pl.BlockSpec((2 * D, K * D), lambda i, j: (0, 0)),
                  pl.BlockSpec((2 * D, K), lambda i, j: (0, 0))],
        out_specs=[pl.BlockSpec((1, 1, BN), lambda i, j: (i, 0, 0)),
                   pl.BlockSpec((1, 1, BN), lambda i, j: (i, 0, 0))],
        out_shape=[jax.ShapeDtypeStruct((PAR, 1, BN), jnp.float32),
                   jax.ShapeDtypeStruct((PAR, 1, BN), jnp.float32)],
        compiler_params=pltpu.CompilerParams(
            dimension_semantics=("parallel", "arbitrary")),
    )(X, At, Bb)

    mv = m[:, 0, 0]
    sv = s[:, 0, 0]
    Mx = jnp.max(mv)
    return -(Mx + jnp.log(jnp.sum(sv * jnp.exp(mv - Mx))))


# speedup vs baseline: 1.2907x; 1.0304x over previous
import jax
import jax.numpy as jnp
import numpy as np
from jax.experimental import pallas as pl
from jax.experimental.pallas import tpu as pltpu

_LOG_2PI = float(np.log(2.0 * np.pi))


def _prep_body(L_ref, mu_ref, w_ref, At_ref, Bb_ref):
    K, D, _ = L_ref.shape
    C = K * D
    r2 = jax.lax.broadcasted_iota(jnp.int32, (D, D), 0)
    c2 = jax.lax.broadcasted_iota(jnp.int32, (D, D), 1)
    tril_m = r2 >= c2
    eye2 = (r2 == c2).astype(jnp.float32)
    sig_list = []
    for k in range(K):
        Lt = jnp.where(tril_m, L_ref[k], 0.0)
        Sig = jax.lax.dot_general(Lt, Lt, (((1,), (1,)), ((), ())),
                                  preferred_element_type=jnp.float32) + eye2
        sig_list.append(Sig)
    M3 = jnp.stack(sig_list, axis=1)

    rowi = jax.lax.broadcasted_iota(jnp.int32, (D, 1, 1), 0)
    coli = jax.lax.broadcasted_iota(jnp.int32, (1, 1, D), 2)

    def body(i, carry):
        M, ld = carry
        rm = rowi == i
        cm = coli == i
        r = jnp.sum(jnp.where(rm, M, 0.0), axis=0, keepdims=True)
        c = jnp.sum(jnp.where(cm, M, 0.0), axis=2, keepdims=True)
        p = jnp.sum(jnp.where(cm, r, 0.0), axis=2, keepdims=True)
        pinv = 1.0 / p
        rp = r * pinv
        Mu = M - c * rp
        rowpatch = jnp.where(cm, pinv, rp)
        Mn = jnp.where(rm, rowpatch, jnp.where(cm, -c * pinv, Mu))
        return Mn, ld + jnp.log(p)

    M3, ld = jax.lax.fori_loop(
        0, D, body, (M3, jnp.zeros((1, K, 1), jnp.float32)))

    A2 = M3.reshape(D, C)
    ld2 = ld.reshape(1, K)
    Bm = jnp.sum(M3 * mu_ref[...].reshape(1, K, D), axis=2)
    BtMu = jax.lax.dot_general(Bm, mu_ref[...], (((0,), (1,)), ((), ())),
                               preferred_element_type=jnp.float32)
    kk1 = jax.lax.broadcasted_iota(jnp.int32, (K, K), 0)
    kk2 = jax.lax.broadcasted_iota(jnp.int32, (K, K), 1)
    cdiag = jnp.sum(jnp.where(kk1 == kk2, BtMu, 0.0), axis=0, keepdims=True)
    w = w_ref[...]
    wm = jnp.max(w)
    logw = w - (wm + jnp.log(jnp.sum(jnp.exp(w - wm))))
    beta = -0.5 * (D * _LOG_2PI + ld2) + logw - 0.5 * cdiag

    At_ref[...] = jnp.concatenate(
        [-0.5 * A2, jnp.zeros((D, C), jnp.float32)], axis=0)
    Bb_ref[...] = jnp.concatenate(
        [Bm, beta, jnp.zeros((D - 1, K), jnp.float32)], axis=0)


def _maha_body(X_ref, At_ref, Bb_ref, m_ref, s_ref):
    j = pl.program_id(1)
    nj = pl.num_programs(1)
    Xb = X_ref[...]
    BN, D = Xb.shape
    KD = At_ref.shape[1]
    K = KD // D
    ones_lane = (jax.lax.broadcasted_iota(jnp.int32, (BN, D), 1) == 0)
    Xaug = jnp.concatenate(
        [Xb, jnp.where(ones_lane, 1.0, 0.0)], axis=1)
    ey = (jax.lax.broadcasted_iota(jnp.int32, (D, D), 0) ==
          jax.lax.broadcasted_iota(jnp.int32, (D, D), 1)).astype(jnp.float32)
    Xt = jax.lax.dot_general(ey, Xb, (((1,), (1,)), ((), ())),
                             preferred_element_type=jnp.float32)
    Tt = jax.lax.dot_general(At_ref[...], Xaug, (((0,), (1,)), ((), ())),
                             preferred_element_type=jnp.float32)
    lb = jax.lax.dot_general(Bb_ref[...], Xaug, (((0,), (1,)), ((), ())),
                             preferred_element_type=jnp.float32)
    Tq = Tt.reshape(K, D, BN)
    qT = jnp.sum(Tq * Xt.reshape(1, D, BN), axis=1)
    logp = lb + qT
    mk = jnp.max(logp, axis=0, keepdims=True)
    ss = jnp.sum(jnp.exp(logp - mk), axis=0, keepdims=True)

    @pl.when(j == 0)
    def _():
        m_ref[...] = mk.reshape(1, 1, BN)
        s_ref[...] = ss.reshape(1, 1, BN)

    @pl.when(j > 0)
    def _():
        mp = m_ref[...].reshape(1, BN)
        sp = s_ref[...].reshape(1, BN)
        mn = jnp.maximum(mp, mk)
        s_ref[...] = (sp * jnp.exp(mp - mn) + ss * jnp.exp(mk - mn)).reshape(1, 1, BN)
        m_ref[...] = mn.reshape(1, 1, BN)

    @pl.when(j == nj - 1)
    def _():
        mv = m_ref[...].reshape(1, BN)
        sv = s_ref[...].reshape(1, BN)
        mtot = jnp.max(mv)
        stot = jnp.sum(sv * jnp.exp(mv - mtot))
        m_ref[...] = jnp.full((1, 1, BN), mtot, jnp.float32)
        s_ref[...] = jnp.full((1, 1, BN), stot, jnp.float32)


def kernel(X, mu, L, weights, it):
    N, D = X.shape
    K = mu.shape[0]
    w2 = weights.reshape(1, K)
    At, Bb = pl.pallas_call(
        _prep_body,
        out_shape=[jax.ShapeDtypeStruct((2 * D, K * D), jnp.float32),
                   jax.ShapeDtypeStruct((2 * D, K), jnp.float32)],
    )(L, mu, w2)

    BN = 2048
    PAR = 8
    SEQ = N // (PAR * BN)
    m, s = pl.pallas_call(
        _maha_body,
        grid=(PAR, SEQ),
        in_specs=[pl.BlockSpec((BN, D), lambda i, j: (i * SEQ + j, 0)),
                  pl.BlockSpec((2 * D, K * D), lambda i, j: (0, 0)),
                  pl.BlockSpec((2 * D, K), lambda i, j: (0, 0))],
        out_specs=[pl.BlockSpec((1, 1, BN), lambda i, j: (i, 0, 0)),
                   pl.BlockSpec((1, 1, BN), lambda i, j: (i, 0, 0))],
        out_shape=[jax.ShapeDtypeStruct((PAR, 1, BN), jnp.float32),
                   jax.ShapeDtypeStruct((PAR, 1, BN), jnp.float32)],
        compiler_params=pltpu.CompilerParams(
            dimension_semantics=("parallel", "arbitrary")),
    )(X, At, Bb)

    mv = m[:, 0, 0]
    sv = s[:, 0, 0]
    Mx = jnp.max(mv)
    return -(Mx + jnp.log(jnp.sum(sv * jnp.exp(mv - Mx))))


# speedup vs baseline: 1.3124x; 1.0168x over previous
import jax
import jax.numpy as jnp
import numpy as np
from jax.experimental import pallas as pl
from jax.experimental.pallas import tpu as pltpu

_LOG_2PI = float(np.log(2.0 * np.pi))


def _prep_body(L_ref, mu_ref, w_ref, At_ref, Bb_ref):
    K, D, _ = L_ref.shape
    C = K * D
    r2 = jax.lax.broadcasted_iota(jnp.int32, (D, D), 0)
    c2 = jax.lax.broadcasted_iota(jnp.int32, (D, D), 1)
    tril_m = r2 >= c2
    eye2 = (r2 == c2).astype(jnp.float32)
    sig_list = []
    for k in range(K):
        Lt = jnp.where(tril_m, L_ref[k], 0.0)
        Sig = jax.lax.dot_general(Lt, Lt, (((1,), (1,)), ((), ())),
                                  preferred_element_type=jnp.float32) + eye2
        sig_list.append(Sig)
    M3 = jnp.stack(sig_list, axis=1)

    rowi = jax.lax.broadcasted_iota(jnp.int32, (D, 1, 1), 0)
    coli = jax.lax.broadcasted_iota(jnp.int32, (1, 1, D), 2)

    def body(i, carry):
        M, ld = carry
        rm = rowi == i
        cm = coli == i
        r = jnp.sum(jnp.where(rm, M, 0.0), axis=0, keepdims=True)
        c = jnp.sum(jnp.where(cm, M, 0.0), axis=2, keepdims=True)
        p = jnp.sum(jnp.where(cm, r, 0.0), axis=2, keepdims=True)
        pinv = 1.0 / p
        rp = r * pinv
        Mu = M - c * rp
        rowpatch = jnp.where(cm, pinv, rp)
        Mn = jnp.where(rm, rowpatch, jnp.where(cm, -c * pinv, Mu))
        return Mn, ld + jnp.log(p)

    M3, ld = jax.lax.fori_loop(
        0, D, body, (M3, jnp.zeros((1, K, 1), jnp.float32)))

    A2 = M3.reshape(D, C)
    ld2 = ld.reshape(1, K)
    Bm = jnp.sum(M3 * mu_ref[...].reshape(1, K, D), axis=2)
    BtMu = jax.lax.dot_general(Bm, mu_ref[...], (((0,), (1,)), ((), ())),
                               preferred_element_type=jnp.float32)
    kk1 = jax.lax.broadcasted_iota(jnp.int32, (K, K), 0)
    kk2 = jax.lax.broadcasted_iota(jnp.int32, (K, K), 1)
    cdiag = jnp.sum(jnp.where(kk1 == kk2, BtMu, 0.0), axis=0, keepdims=True)
    w = w_ref[...]
    wm = jnp.max(w)
    logw = w - (wm + jnp.log(jnp.sum(jnp.exp(w - wm))))
    beta = -0.5 * (D * _LOG_2PI + ld2) + logw - 0.5 * cdiag

    At_ref[...] = jnp.concatenate(
        [-0.5 * A2, jnp.zeros((D, C), jnp.float32)], axis=0)
    Bb_ref[...] = jnp.concatenate(
        [Bm, beta, jnp.zeros((D - 1, K), jnp.float32)], axis=0)


def _maha_body(X_ref, At_ref, Bb_ref, m_ref, s_ref):
    j = pl.program_id(1)
    nj = pl.num_programs(1)
    Xb = X_ref[...]
    BN, D = Xb.shape
    KD = At_ref.shape[1]
    K = KD // D
    ones_lane = (jax.lax.broadcasted_iota(jnp.int32, (BN, D), 1) == 0)
    Xaug = jnp.concatenate(
        [Xb, jnp.where(ones_lane, 1.0, 0.0)], axis=1)
    ey = (jax.lax.broadcasted_iota(jnp.int32, (D, D), 0) ==
          jax.lax.broadcasted_iota(jnp.int32, (D, D), 1)).astype(jnp.float32)
    Xt = jax.lax.dot_general(ey, Xb, (((1,), (1,)), ((), ())),
                             preferred_element_type=jnp.float32)
    Tt = jax.lax.dot_general(At_ref[...], Xaug, (((0,), (1,)), ((), ())),
                             preferred_element_type=jnp.float32)
    lb = jax.lax.dot_general(Bb_ref[...], Xaug, (((0,), (1,)), ((), ())),
                             preferred_element_type=jnp.float32)
    Tq = Tt.reshape(K, D, BN)
    qT = jnp.sum(Tq * Xt.reshape(1, D, BN), axis=1)
    logp = lb + qT
    mk = jnp.max(logp, axis=0, keepdims=True)
    ss = jnp.sum(jnp.exp(logp - mk), axis=0, keepdims=True)

    @pl.when(j == 0)
    def _():
        m_ref[...] = mk.reshape(1, 1, BN)
        s_ref[...] = ss.reshape(1, 1, BN)

    @pl.when(j > 0)
    def _():
        mp = m_ref[...].reshape(1, BN)
        sp = s_ref[...].reshape(1, BN)
        mn = jnp.maximum(mp, mk)
        s_ref[...] = (sp * jnp.exp(mp - mn) + ss * jnp.exp(mk - mn)).reshape(1, 1, BN)
        m_ref[...] = mn.reshape(1, 1, BN)

    @pl.when(j == nj - 1)
    def _():
        mv = m_ref[...].reshape(1, BN)
        sv = s_ref[...].reshape(1, BN)
        mtot = jnp.max(mv)
        stot = jnp.sum(sv * jnp.exp(mv - mtot))
        m_ref[...] = jnp.full((1, 1, BN), mtot, jnp.float32)
        s_ref[...] = jnp.full((1, 1, BN), stot, jnp.float32)


def kernel(X, mu, L, weights, it):
    N, D = X.shape
    K = mu.shape[0]
    w2 = weights.reshape(1, K)
    At, Bb = pl.pallas_call(
        _prep_body,
        out_shape=[jax.ShapeDtypeStruct((2 * D, K * D), jnp.float32),
                   jax.ShapeDtypeStruct((2 * D, K), jnp.float32)],
    )(L, mu, w2)

    BN = 4096
    PAR = 8
    SEQ = N // (PAR * BN)
    m, s = pl.pallas_call(
        _maha_body,
        grid=(PAR, SEQ),
        in_specs=[pl.BlockSpec((BN, D), lambda i, j: (i * SEQ + j, 0)),
                  pl.BlockSpec((2 * D, K * D), lambda i, j: (0, 0)),
                  pl.BlockSpec((2 * D, K), lambda i, j: (0, 0))],
        out_specs=[pl.BlockSpec((1, 1, BN), lambda i, j: (i, 0, 0)),
                   pl.BlockSpec((1, 1, BN), lambda i, j: (i, 0, 0))],
        out_shape=[jax.ShapeDtypeStruct((PAR, 1, BN), jnp.float32),
                   jax.ShapeDtypeStruct((PAR, 1, BN), jnp.float32)],
        compiler_params=pltpu.CompilerParams(
            dimension_semantics=("parallel", "arbitrary")),
    )(X, At, Bb)

    mv = m[:, 0, 0]
    sv = s[:, 0, 0]
    Mx = jnp.max(mv)
    return -(Mx + jnp.log(jnp.sum(sv * jnp.exp(mv - Mx))))
